# baseline (speedup 1.0000x reference)
import functools

import jax
import jax.numpy as jnp
from jax import lax
from jax.experimental import pallas as pl
from jax.experimental.pallas import tpu as pltpu
from jax.experimental.pallas import tpu_sc as plsc

NU = 5000
NI = 5000
R = 5
IN_FEAT = 128
HID = 128
HALF = 64
QCOL = 32
NQ = IN_FEAT // QCOL

NC = 2
NS = 16
CHUNK = 128

ACC_ROWS = R * NU + 88
ROWS_PT = ACC_ROWS // NS


def _sc_mesh():
    return plsc.VectorSubcoreMesh(
        core_axis_name="c", subcore_axis_name="s", num_cores=NC, num_subcores=NS)


NBUF = 8
LAG = 4


def _sc_accumulate(xi_qs, xu_qs, gu, gv, ku, kv, ones_i, zf):
    nchunk = gu.shape[1]
    nq = len(xi_qs)

    def body(*refs):
        xi_t = refs[0:nq]
        xu_t = refs[nq:2 * nq]
        gu_h, gv_h, ku_h, kv_h, ones_h, zf_h = refs[2 * nq:2 * nq + 6]
        ou, ouc, ov, ovc = refs[2 * nq + 6:2 * nq + 10]
        (keys_t, gidx_t, rows_t, ones_t, acc_f,
         sem_g, sem_s) = refs[2 * nq + 10:]
        cid = lax.axis_index("c")
        sid = lax.axis_index("s")
        base = sid * ROWS_PT
        tail_rows = R * NU - (NS - 1) * ROWS_PT

        def dump(src, dst_cols):
            @pl.when(sid < NS - 1)
            def _():
                pltpu.sync_copy(src.at[pl.ds(base, ROWS_PT)],
                                dst_cols(pl.ds(base, ROWS_PT)))

            @pl.when(sid == NS - 1)
            def _():
                pltpu.sync_copy(src.at[pl.ds(base, tail_rows)],
                                dst_cols(pl.ds(base, tail_rows)))

        def run_side(tabs, gidx_hbm, keys_hbm, o, oc):
            pltpu.sync_copy(keys_hbm.at[sid], keys_t)
            pltpu.sync_copy(gidx_hbm.at[sid], gidx_t)
            pltpu.sync_copy(ones_h, ones_t)

            pltpu.sync_copy(zf_h, acc_f.at[pl.ds(base, ROWS_PT)])
            plsc.subcore_barrier()

            def cchunk(j, carry):
                pltpu.async_copy(ones_t, acc_f.at[keys_t.at[j]], sem_s, add=True)

                @pl.when(j >= LAG)
                def _():
                    pltpu.make_async_copy(
                        ones_t, acc_f.at[keys_t.at[0]], sem_s).wait()

                return carry

            lax.fori_loop(0, nchunk, cchunk, 0)
            for _ in range(LAG):
                pltpu.make_async_copy(
                    ones_t, acc_f.at[keys_t.at[0]], sem_s).wait()
            plsc.subcore_barrier()
            dump(acc_f, lambda rs: oc.at[rs])
            plsc.subcore_barrier()

            for q in range(nq):
                tab = tabs[q]
                pltpu.sync_copy(zf_h, acc_f.at[pl.ds(base, ROWS_PT)])
                plsc.subcore_barrier()

                for b in range(LAG):
                    pltpu.async_copy(tab.at[gidx_t.at[b]], rows_t.at[b], sem_g)

                def chunk(j, carry):
                    slot = lax.rem(j, NBUF)
                    pltpu.make_async_copy(
                        tab.at[gidx_t.at[j]], rows_t.at[slot], sem_g).wait()
                    pltpu.async_copy(
                        rows_t.at[slot], acc_f.at[keys_t.at[j]], sem_s, add=True)

                    @pl.when(j + LAG < nchunk)
                    def _():
                        @pl.when(j >= LAG)
                        def _():
                            pltpu.make_async_copy(
                                rows_t.at[0], acc_f.at[keys_t.at[0]],
                                sem_s).wait()

                        pltpu.async_copy(
                            tab.at[gidx_t.at[j + LAG]],
                            rows_t.at[lax.rem(j + LAG, NBUF)], sem_g)

                    return carry

                lax.fori_loop(0, nchunk, chunk, 0)
                for b in range(NBUF):
                    pltpu.make_async_copy(
                        rows_t.at[b], acc_f.at[keys_t.at[0]], sem_s).wait()
                plsc.subcore_barrier()
                dump(acc_f, lambda rs, q=q: o.at[rs, pl.ds(q * QCOL, QCOL)])
                plsc.subcore_barrier()

        @pl.when(cid == 0)
        def _():
            run_side(xi_t, gu_h, ku_h, ou, ouc)

        @pl.when(cid == 1)
        def _():
            run_side(xu_t, gv_h, kv_h, ov, ovc)

    f = pl.kernel(
        body,
        out_type=[
            jax.ShapeDtypeStruct((R * NU, IN_FEAT), jnp.float32),
            jax.ShapeDtypeStruct((R * NU, QCOL), jnp.float32),
            jax.ShapeDtypeStruct((R * NI, IN_FEAT), jnp.float32),
            jax.ShapeDtypeStruct((R * NI, QCOL), jnp.float32),
        ],
        mesh=_sc_mesh(),
        scratch_types=[
            pltpu.VMEM((nchunk, CHUNK), jnp.int32),
            pltpu.VMEM((nchunk, CHUNK), jnp.int32),
            pltpu.VMEM((NBUF, CHUNK, QCOL), jnp.float32),
            pltpu.VMEM((CHUNK, QCOL), jnp.float32),
            pltpu.VMEM_SHARED((ACC_ROWS, QCOL), jnp.float32),
            pltpu.SemaphoreType.DMA,
            pltpu.SemaphoreType.DMA,
        ],
        compiler_params=pltpu.CompilerParams(use_tc_tiling_on_sc=False),
    )
    return f(*xi_qs, *xu_qs, gu, gv, ku, kv, ones_i, zf)


def _tc_body(f_ref, cnt_ref, w_ref, out_ref):
    r = pl.program_id(1)
    c = cnt_ref[:, 0:1]
    rc = jnp.where(c > 0.5, 1.0 / c, 0.0)
    part = jnp.dot(f_ref[...] * rc, w_ref[0],
                   preferred_element_type=jnp.float32)

    @pl.when(r == 0)
    def _():
        out_ref[...] = part

    @pl.when(r > 0)
    def _():
        out_ref[...] += part


def _tc_project(f, cnt, weight, n):
    blk = 1000
    nblk = n // blk
    return pl.pallas_call(
        _tc_body,
        grid=(nblk, R),
        in_specs=[
            pl.BlockSpec((blk, IN_FEAT), lambda i, r: (r * nblk + i, 0)),
            pl.BlockSpec((blk, QCOL), lambda i, r: (r * nblk + i, 0)),
            pl.BlockSpec((1, IN_FEAT, HID), lambda i, r: (r, 0, 0)),
        ],
        out_specs=pl.BlockSpec((blk, HID), lambda i, r: (i, 0)),
        out_shape=jax.ShapeDtypeStruct((n, HID), jnp.float32),
    )(f, cnt, weight)


def kernel(x_user, x_item, u_s, v_s, rate, weight):
    E = u_s.shape[0]
    nchunk = -(-E // (NS * CHUNK))
    e_pad = NS * nchunk * CHUNK
    pad = e_pad - E

    u_s = u_s.astype(jnp.int32)
    v_s = v_s.astype(jnp.int32)
    rate = rate.astype(jnp.int32)

    key_u = rate * NU + u_s
    key_v = rate * NI + v_s
    trash = jnp.full((pad,), R * NU, jnp.int32)
    zero_idx = jnp.zeros((pad,), jnp.int32)
    ku = jnp.concatenate([key_u, trash]).reshape(NS, nchunk, CHUNK)
    kv = jnp.concatenate([key_v, trash]).reshape(NS, nchunk, CHUNK)
    gu = jnp.concatenate([v_s, zero_idx]).reshape(NS, nchunk, CHUNK)
    gv = jnp.concatenate([u_s, zero_idx]).reshape(NS, nchunk, CHUNK)

    xi_qs = [x_item[:, q * QCOL:(q + 1) * QCOL] for q in range(NQ)]
    xu_qs = [x_user[:, q * QCOL:(q + 1) * QCOL] for q in range(NQ)]
    ones_i = jnp.ones((CHUNK, QCOL), jnp.float32)
    zf = jnp.zeros((ROWS_PT, QCOL), jnp.float32)

    ou, ouc, ov, ovc = _sc_accumulate(
        xi_qs, xu_qs, gu, gv, ku, kv, ones_i, zf)

    h_u = _tc_project(ou, ouc, weight, NU)
    h_v = _tc_project(ov, ovc, weight, NI)
    return h_u, h_v


# speedup vs baseline: 1.0450x; 1.0450x over previous
import functools

import jax
import jax.numpy as jnp
from jax import lax
from jax.experimental import pallas as pl
from jax.experimental.pallas import tpu as pltpu
from jax.experimental.pallas import tpu_sc as plsc

NU = 5000
NI = 5000
R = 5
IN_FEAT = 128
HID = 128
HALF = 64
QCOL = 32
NQ = IN_FEAT // QCOL

NC = 2
NS = 16
CHUNK = 128

ACC_ROWS = R * NU + 88
ROWS_PT = ACC_ROWS // NS


def _sc_mesh():
    return plsc.VectorSubcoreMesh(
        core_axis_name="c", subcore_axis_name="s", num_cores=NC, num_subcores=NS)


NBUF = 8
LAG = 4


def _sc_accumulate(xi_qs, xu_qs, gu, gv, ku, kv, ones_i, zf):
    nchunk = gu.shape[1]
    nq = len(xi_qs)

    def body(*refs):
        xi_t = refs[0:nq]
        xu_t = refs[nq:2 * nq]
        gu_h, gv_h, ku_h, kv_h, ones_h, zf_h = refs[2 * nq:2 * nq + 6]
        ou, ouc, ov, ovc = refs[2 * nq + 6:2 * nq + 10]
        (keys_t, gidx_t, rows_t, ones_t, acc_f,
         sem_g, sem_s) = refs[2 * nq + 10:]
        cid = lax.axis_index("c")
        sid = lax.axis_index("s")
        base = sid * ROWS_PT
        tail_rows = R * NU - (NS - 1) * ROWS_PT

        def dump(src, dst_cols):
            @pl.when(sid < NS - 1)
            def _():
                pltpu.sync_copy(src.at[pl.ds(base, ROWS_PT)],
                                dst_cols(pl.ds(base, ROWS_PT)))

            @pl.when(sid == NS - 1)
            def _():
                pltpu.sync_copy(src.at[pl.ds(base, tail_rows)],
                                dst_cols(pl.ds(base, tail_rows)))

        def run_side(tabs, gidx_hbm, keys_hbm, o, oc):
            pltpu.sync_copy(keys_hbm.at[sid], keys_t)
            pltpu.sync_copy(gidx_hbm.at[sid], gidx_t)
            pltpu.sync_copy(ones_h, ones_t)

            scope_cnt = jax.named_scope("cnt_pass")
            scope_cnt.__enter__()
            pltpu.sync_copy(zf_h, acc_f.at[pl.ds(base, ROWS_PT)])
            plsc.subcore_barrier()

            def cchunk(j, carry):
                pltpu.async_copy(ones_t, acc_f.at[keys_t.at[j]], sem_s, add=True)

                @pl.when(j >= LAG)
                def _():
                    pltpu.make_async_copy(
                        ones_t, acc_f.at[keys_t.at[0]], sem_s).wait()

                return carry

            lax.fori_loop(0, nchunk, cchunk, 0)
            for _ in range(LAG):
                pltpu.make_async_copy(
                    ones_t, acc_f.at[keys_t.at[0]], sem_s).wait()
            plsc.subcore_barrier()
            dump(acc_f, lambda rs: oc.at[rs])
            plsc.subcore_barrier()
            scope_cnt.__exit__(None, None, None)

            for q in range(nq):
                scope_q = jax.named_scope(f"feat_pass{q}")
                scope_q.__enter__()
                tab = tabs[q]
                pltpu.sync_copy(zf_h, acc_f.at[pl.ds(base, ROWS_PT)])
                plsc.subcore_barrier()

                for b in range(LAG):
                    pltpu.async_copy(tab.at[gidx_t.at[b]], rows_t.at[b], sem_g)

                def chunk(j, carry):
                    slot = lax.rem(j, NBUF)
                    pltpu.make_async_copy(
                        tab.at[gidx_t.at[j]], rows_t.at[slot], sem_g).wait()
                    pltpu.async_copy(
                        rows_t.at[slot], acc_f.at[keys_t.at[j]], sem_s, add=True)

                    @pl.when(j + LAG < nchunk)
                    def _():
                        @pl.when(j >= LAG)
                        def _():
                            pltpu.make_async_copy(
                                rows_t.at[0], acc_f.at[keys_t.at[0]],
                                sem_s).wait()

                        pltpu.async_copy(
                            tab.at[gidx_t.at[j + LAG]],
                            rows_t.at[lax.rem(j + LAG, NBUF)], sem_g)

                    return carry

                lax.fori_loop(0, nchunk, chunk, 0)
                for b in range(NBUF):
                    pltpu.make_async_copy(
                        rows_t.at[b], acc_f.at[keys_t.at[0]], sem_s).wait()
                plsc.subcore_barrier()
                dump(acc_f, lambda rs, q=q: o.at[rs, pl.ds(q * QCOL, QCOL)])
                plsc.subcore_barrier()
                scope_q.__exit__(None, None, None)

        @pl.when(cid == 0)
        def _():
            run_side(xi_t, gu_h, ku_h, ou, ouc)

        @pl.when(cid == 1)
        def _():
            run_side(xu_t, gv_h, kv_h, ov, ovc)

    f = pl.kernel(
        body,
        out_type=[
            jax.ShapeDtypeStruct((R * NU, IN_FEAT), jnp.float32),
            jax.ShapeDtypeStruct((R * NU, QCOL), jnp.float32),
            jax.ShapeDtypeStruct((R * NI, IN_FEAT), jnp.float32),
            jax.ShapeDtypeStruct((R * NI, QCOL), jnp.float32),
        ],
        mesh=_sc_mesh(),
        scratch_types=[
            pltpu.VMEM((nchunk, CHUNK), jnp.int32),
            pltpu.VMEM((nchunk, CHUNK), jnp.int32),
            pltpu.VMEM((NBUF, CHUNK, QCOL), jnp.float32),
            pltpu.VMEM((CHUNK, QCOL), jnp.float32),
            pltpu.VMEM_SHARED((ACC_ROWS, QCOL), jnp.float32),
            pltpu.SemaphoreType.DMA,
            pltpu.SemaphoreType.DMA,
        ],
        compiler_params=pltpu.CompilerParams(use_tc_tiling_on_sc=False),
    )
    return f(*xi_qs, *xu_qs, gu, gv, ku, kv, ones_i, zf)


def _tc_body(f_ref, cnt_ref, w_ref, out_ref):
    acc = jnp.zeros(out_ref.shape, jnp.float32)
    for r in range(R):
        c = cnt_ref[r, :, 0:1]
        rc = jnp.where(c > 0.5, 1.0 / c, 0.0)
        acc = acc + jnp.dot(f_ref[r] * rc, w_ref[r],
                            preferred_element_type=jnp.float32)
    out_ref[...] = acc


def _tc_project(f, cnt, weight, n):
    blk = 1000
    grid = (n // blk,)
    return pl.pallas_call(
        _tc_body,
        grid=grid,
        in_specs=[
            pl.BlockSpec((R, blk, IN_FEAT), lambda g: (0, g, 0)),
            pl.BlockSpec((R, blk, QCOL), lambda g: (0, g, 0)),
            pl.BlockSpec((R, IN_FEAT, HID), lambda g: (0, 0, 0)),
        ],
        out_specs=pl.BlockSpec((blk, HID), lambda g: (g, 0)),
        out_shape=jax.ShapeDtypeStruct((n, HID), jnp.float32),
    )(f, cnt, weight)


def kernel(x_user, x_item, u_s, v_s, rate, weight):
    E = u_s.shape[0]
    nchunk = -(-E // (NS * CHUNK))
    e_pad = NS * nchunk * CHUNK
    pad = e_pad - E

    u_s = u_s.astype(jnp.int32)
    v_s = v_s.astype(jnp.int32)
    rate = rate.astype(jnp.int32)

    key_u = rate * NU + u_s
    key_v = rate * NI + v_s
    trash = jnp.full((pad,), R * NU, jnp.int32)
    zero_idx = jnp.zeros((pad,), jnp.int32)
    ku = jnp.concatenate([key_u, trash]).reshape(NS, nchunk, CHUNK)
    kv = jnp.concatenate([key_v, trash]).reshape(NS, nchunk, CHUNK)
    gu = jnp.concatenate([v_s, zero_idx]).reshape(NS, nchunk, CHUNK)
    gv = jnp.concatenate([u_s, zero_idx]).reshape(NS, nchunk, CHUNK)

    xi_qs = [x_item[:, q * QCOL:(q + 1) * QCOL] for q in range(NQ)]
    xu_qs = [x_user[:, q * QCOL:(q + 1) * QCOL] for q in range(NQ)]
    ones_i = jnp.ones((CHUNK, QCOL), jnp.float32)
    zf = jnp.zeros((ROWS_PT, QCOL), jnp.float32)

    ou, ouc, ov, ovc = _sc_accumulate(
        xi_qs, xu_qs, gu, gv, ku, kv, ones_i, zf)

    h_u = _tc_project(ou.reshape(R, NU, IN_FEAT), ouc.reshape(R, NU, QCOL),
                      weight, NU)
    h_v = _tc_project(ov.reshape(R, NI, IN_FEAT), ovc.reshape(R, NI, QCOL),
                      weight, NI)
    return h_u, h_v


# speedup vs baseline: 1.1472x; 1.0977x over previous
import functools

import jax
import jax.numpy as jnp
from jax import lax
from jax.experimental import pallas as pl
from jax.experimental.pallas import tpu as pltpu
from jax.experimental.pallas import tpu_sc as plsc

NU = 5000
NI = 5000
R = 5
IN_FEAT = 128
HID = 128
HALF = 64
QCOL = 32
NQ = IN_FEAT // QCOL

NC = 2
NS = 16
CHUNK = 128

ACC_ROWS = R * NU + 88
ROWS_PT = ACC_ROWS // NS


def _sc_mesh():
    return plsc.VectorSubcoreMesh(
        core_axis_name="c", subcore_axis_name="s", num_cores=NC, num_subcores=NS)


NBUF = 6
LAG = 3


def _sc_accumulate(xi_qs, xu_qs, gu, gv, ku, kv, ones_i, zf):
    nchunk = gu.shape[1]
    nq = len(xi_qs)

    def body(*refs):
        xi_t = refs[0:nq]
        xu_t = refs[nq:2 * nq]
        gu_h, gv_h, ku_h, kv_h, ones_h, zf_h = refs[2 * nq:2 * nq + 6]
        ou, ouc, ov, ovc = refs[2 * nq + 6:2 * nq + 10]
        (keys_t, gidx_t, rows_t, ones_t, acc_f, tab_sp,
         sem_g, sem_s) = refs[2 * nq + 10:]
        cid = lax.axis_index("c")
        sid = lax.axis_index("s")
        base = sid * ROWS_PT
        tail_rows = R * NU - (NS - 1) * ROWS_PT

        def dump(src, dst_cols):
            @pl.when(sid < NS - 1)
            def _():
                pltpu.sync_copy(src.at[pl.ds(base, ROWS_PT)],
                                dst_cols(pl.ds(base, ROWS_PT)))

            @pl.when(sid == NS - 1)
            def _():
                pltpu.sync_copy(src.at[pl.ds(base, tail_rows)],
                                dst_cols(pl.ds(base, tail_rows)))

        def run_side(tabs, gidx_hbm, keys_hbm, o, oc):
            pltpu.sync_copy(keys_hbm.at[sid], keys_t)
            pltpu.sync_copy(gidx_hbm.at[sid], gidx_t)
            pltpu.sync_copy(ones_h, ones_t)

            scope_cnt = jax.named_scope("cnt_pass")
            scope_cnt.__enter__()
            pltpu.sync_copy(zf_h, acc_f.at[pl.ds(base, ROWS_PT)])
            plsc.subcore_barrier()

            def cchunk(j, carry):
                pltpu.async_copy(ones_t, acc_f.at[keys_t.at[j]], sem_s, add=True)

                @pl.when(j >= LAG)
                def _():
                    pltpu.make_async_copy(
                        ones_t, acc_f.at[keys_t.at[0]], sem_s).wait()

                return carry

            lax.fori_loop(0, nchunk, cchunk, 0)
            for _ in range(LAG):
                pltpu.make_async_copy(
                    ones_t, acc_f.at[keys_t.at[0]], sem_s).wait()
            plsc.subcore_barrier()
            dump(acc_f, lambda rs: oc.at[rs])
            plsc.subcore_barrier()
            scope_cnt.__exit__(None, None, None)

            for q in range(nq):
                scope_q = jax.named_scope(f"feat_pass{q}")
                scope_q.__enter__()
                tab = tab_sp
                pltpu.sync_copy(zf_h, acc_f.at[pl.ds(base, ROWS_PT)])

                @pl.when(sid == 0)
                def _(q=q):
                    pltpu.sync_copy(tabs[q], tab_sp)

                plsc.subcore_barrier()

                for b in range(LAG):
                    pltpu.async_copy(tab.at[gidx_t.at[b]], rows_t.at[b], sem_g)

                def chunk(j, carry):
                    slot = lax.rem(j, NBUF)
                    pltpu.make_async_copy(
                        tab.at[gidx_t.at[j]], rows_t.at[slot], sem_g).wait()
                    pltpu.async_copy(
                        rows_t.at[slot], acc_f.at[keys_t.at[j]], sem_s, add=True)

                    @pl.when(j + LAG < nchunk)
                    def _():
                        @pl.when(j >= LAG)
                        def _():
                            pltpu.make_async_copy(
                                rows_t.at[0], acc_f.at[keys_t.at[0]],
                                sem_s).wait()

                        pltpu.async_copy(
                            tab.at[gidx_t.at[j + LAG]],
                            rows_t.at[lax.rem(j + LAG, NBUF)], sem_g)

                    return carry

                lax.fori_loop(0, nchunk, chunk, 0)
                for b in range(NBUF):
                    pltpu.make_async_copy(
                        rows_t.at[b], acc_f.at[keys_t.at[0]], sem_s).wait()
                plsc.subcore_barrier()
                dump(acc_f, lambda rs, q=q: o.at[rs, pl.ds(q * QCOL, QCOL)])
                plsc.subcore_barrier()
                scope_q.__exit__(None, None, None)

        @pl.when(cid == 0)
        def _():
            run_side(xi_t, gu_h, ku_h, ou, ouc)

        @pl.when(cid == 1)
        def _():
            run_side(xu_t, gv_h, kv_h, ov, ovc)

    f = pl.kernel(
        body,
        out_type=[
            jax.ShapeDtypeStruct((R * NU, IN_FEAT), jnp.float32),
            jax.ShapeDtypeStruct((R * NU, QCOL), jnp.float32),
            jax.ShapeDtypeStruct((R * NI, IN_FEAT), jnp.float32),
            jax.ShapeDtypeStruct((R * NI, QCOL), jnp.float32),
        ],
        mesh=_sc_mesh(),
        scratch_types=[
            pltpu.VMEM((nchunk, CHUNK), jnp.int32),
            pltpu.VMEM((nchunk, CHUNK), jnp.int32),
            pltpu.VMEM((NBUF, CHUNK, QCOL), jnp.float32),
            pltpu.VMEM((CHUNK, QCOL), jnp.float32),
            pltpu.VMEM_SHARED((ACC_ROWS, QCOL), jnp.float32),
            pltpu.VMEM_SHARED((NU, QCOL), jnp.float32),
            pltpu.SemaphoreType.DMA,
            pltpu.SemaphoreType.DMA,
        ],
        compiler_params=pltpu.CompilerParams(use_tc_tiling_on_sc=False),
    )
    return f(*xi_qs, *xu_qs, gu, gv, ku, kv, ones_i, zf)


def _tc_body(f_ref, cnt_ref, w_ref, out_ref):
    acc = jnp.zeros(out_ref.shape, jnp.float32)
    for r in range(R):
        c = cnt_ref[r, :, 0:1]
        rc = jnp.where(c > 0.5, 1.0 / c, 0.0)
        acc = acc + jnp.dot(f_ref[r] * rc, w_ref[r],
                            preferred_element_type=jnp.float32)
    out_ref[...] = acc


def _tc_project(f, cnt, weight, n):
    blk = 1000
    grid = (n // blk,)
    return pl.pallas_call(
        _tc_body,
        grid=grid,
        in_specs=[
            pl.BlockSpec((R, blk, IN_FEAT), lambda g: (0, g, 0)),
            pl.BlockSpec((R, blk, QCOL), lambda g: (0, g, 0)),
            pl.BlockSpec((R, IN_FEAT, HID), lambda g: (0, 0, 0)),
        ],
        out_specs=pl.BlockSpec((blk, HID), lambda g: (g, 0)),
        out_shape=jax.ShapeDtypeStruct((n, HID), jnp.float32),
    )(f, cnt, weight)


def kernel(x_user, x_item, u_s, v_s, rate, weight):
    E = u_s.shape[0]
    nchunk = -(-E // (NS * CHUNK))
    e_pad = NS * nchunk * CHUNK
    pad = e_pad - E

    u_s = u_s.astype(jnp.int32)
    v_s = v_s.astype(jnp.int32)
    rate = rate.astype(jnp.int32)

    key_u = rate * NU + u_s
    key_v = rate * NI + v_s
    trash = jnp.full((pad,), R * NU, jnp.int32)
    zero_idx = jnp.zeros((pad,), jnp.int32)
    ku = jnp.concatenate([key_u, trash]).reshape(NS, nchunk, CHUNK)
    kv = jnp.concatenate([key_v, trash]).reshape(NS, nchunk, CHUNK)
    gu = jnp.concatenate([v_s, zero_idx]).reshape(NS, nchunk, CHUNK)
    gv = jnp.concatenate([u_s, zero_idx]).reshape(NS, nchunk, CHUNK)

    xi_qs = [x_item[:, q * QCOL:(q + 1) * QCOL] for q in range(NQ)]
    xu_qs = [x_user[:, q * QCOL:(q + 1) * QCOL] for q in range(NQ)]
    ones_i = jnp.ones((CHUNK, QCOL), jnp.float32)
    zf = jnp.zeros((ROWS_PT, QCOL), jnp.float32)

    ou, ouc, ov, ovc = _sc_accumulate(
        xi_qs, xu_qs, gu, gv, ku, kv, ones_i, zf)

    h_u = _tc_project(ou.reshape(R, NU, IN_FEAT), ouc.reshape(R, NU, QCOL),
                      weight, NU)
    h_v = _tc_project(ov.reshape(R, NI, IN_FEAT), ovc.reshape(R, NI, QCOL),
                      weight, NI)
    return h_u, h_v


# speedup vs baseline: 1.1599x; 1.0111x over previous
import functools

import jax
import jax.numpy as jnp
from jax import lax
from jax.experimental import pallas as pl
from jax.experimental.pallas import tpu as pltpu
from jax.experimental.pallas import tpu_sc as plsc

NU = 5000
NI = 5000
R = 5
IN_FEAT = 128
HID = 128
HALF = 64
QCOL = 32
NQ = IN_FEAT // QCOL

NC = 2
NS = 16
CHUNK = 128

ACC_ROWS = R * NU + 88
ROWS_PT = ACC_ROWS // NS


def _sc_mesh():
    return plsc.VectorSubcoreMesh(
        core_axis_name="c", subcore_axis_name="s", num_cores=NC, num_subcores=NS)


NBUF = 6
LAG = 3


def _sc_accumulate(xi_qs, xu_qs, gu, gv, ku, kv, ones_i, zf):
    nchunk = gu.shape[1]
    nq = len(xi_qs)

    def body(*refs):
        xi_t = refs[0:nq]
        xu_t = refs[nq:2 * nq]
        gu_h, gv_h, ku_h, kv_h, ones_h, zf_h = refs[2 * nq:2 * nq + 6]
        ou, ouc, ov, ovc = refs[2 * nq + 6:2 * nq + 10]
        (keys_t, gidx_t, rows_t, ones_t, acc_f, tab_sp,
         sem_g, sem_s) = refs[2 * nq + 10:]
        cid = lax.axis_index("c")
        sid = lax.axis_index("s")
        base = sid * ROWS_PT
        tail_rows = R * NU - (NS - 1) * ROWS_PT

        def dump(src, dst_cols):
            @pl.when(sid < NS - 1)
            def _():
                pltpu.sync_copy(src.at[pl.ds(base, ROWS_PT)],
                                dst_cols(pl.ds(base, ROWS_PT)))

            @pl.when(sid == NS - 1)
            def _():
                pltpu.sync_copy(src.at[pl.ds(base, tail_rows)],
                                dst_cols(pl.ds(base, tail_rows)))

        def run_side(tabs, gidx_hbm, keys_hbm, o, oc):
            pltpu.sync_copy(keys_hbm.at[sid], keys_t)
            pltpu.sync_copy(gidx_hbm.at[sid], gidx_t)
            pltpu.sync_copy(ones_h, ones_t)

            scope_cnt = jax.named_scope("cnt_pass")
            scope_cnt.__enter__()
            pltpu.sync_copy(zf_h, acc_f.at[pl.ds(base, ROWS_PT)])
            plsc.subcore_barrier()

            def cchunk(j, carry):
                pltpu.async_copy(ones_t, acc_f.at[keys_t.at[j]], sem_s, add=True)

                @pl.when(j >= LAG)
                def _():
                    pltpu.make_async_copy(
                        ones_t, acc_f.at[keys_t.at[0]], sem_s).wait()

                return carry

            lax.fori_loop(0, nchunk, cchunk, 0)
            for _ in range(LAG):
                pltpu.make_async_copy(
                    ones_t, acc_f.at[keys_t.at[0]], sem_s).wait()
            plsc.subcore_barrier()
            dump(acc_f, lambda rs: oc.at[rs])
            plsc.subcore_barrier()
            scope_cnt.__exit__(None, None, None)

            for q in range(nq):
                scope_q = jax.named_scope(f"feat_pass{q}")
                scope_q.__enter__()
                tab = tab_sp
                pltpu.sync_copy(zf_h, acc_f.at[pl.ds(base, ROWS_PT)])

                @pl.when(sid == 0)
                def _(q=q):
                    pltpu.sync_copy(tabs[q], tab_sp)

                plsc.subcore_barrier()

                for b in range(LAG):
                    pltpu.async_copy(tab.at[gidx_t.at[b]], rows_t.at[b], sem_g)

                def chunk(j, carry):
                    slot = lax.rem(j, NBUF)
                    pltpu.make_async_copy(
                        tab.at[gidx_t.at[j]], rows_t.at[slot], sem_g).wait()
                    pltpu.async_copy(
                        rows_t.at[slot], acc_f.at[keys_t.at[j]], sem_s, add=True)

                    @pl.when(j + LAG < nchunk)
                    def _():
                        @pl.when(j >= LAG)
                        def _():
                            pltpu.make_async_copy(
                                rows_t.at[0], acc_f.at[keys_t.at[0]],
                                sem_s).wait()

                        pltpu.async_copy(
                            tab.at[gidx_t.at[j + LAG]],
                            rows_t.at[lax.rem(j + LAG, NBUF)], sem_g)

                    return carry

                lax.fori_loop(0, nchunk, chunk, 0)
                for b in range(NBUF):
                    pltpu.make_async_copy(
                        rows_t.at[b], acc_f.at[keys_t.at[0]], sem_s).wait()
                plsc.subcore_barrier()
                dump(acc_f, lambda rs, q=q: o.at[rs, pl.ds(q * QCOL, QCOL)])
                plsc.subcore_barrier()
                scope_q.__exit__(None, None, None)

        @pl.when(cid == 0)
        def _():
            run_side(xi_t, gu_h, ku_h, ou, ouc)

        @pl.when(cid == 1)
        def _():
            run_side(xu_t, gv_h, kv_h, ov, ovc)

    f = pl.kernel(
        body,
        out_type=[
            jax.ShapeDtypeStruct((R * NU, IN_FEAT), jnp.float32),
            jax.ShapeDtypeStruct((R * NU, QCOL), jnp.float32),
            jax.ShapeDtypeStruct((R * NI, IN_FEAT), jnp.float32),
            jax.ShapeDtypeStruct((R * NI, QCOL), jnp.float32),
        ],
        mesh=_sc_mesh(),
        scratch_types=[
            pltpu.VMEM((nchunk, CHUNK), jnp.int32),
            pltpu.VMEM((nchunk, CHUNK), jnp.int32),
            pltpu.VMEM((NBUF, CHUNK, QCOL), jnp.float32),
            pltpu.VMEM((CHUNK, QCOL), jnp.float32),
            pltpu.VMEM_SHARED((ACC_ROWS, QCOL), jnp.float32),
            pltpu.VMEM_SHARED((NU, QCOL), jnp.float32),
            pltpu.SemaphoreType.DMA,
            pltpu.SemaphoreType.DMA,
        ],
        compiler_params=pltpu.CompilerParams(use_tc_tiling_on_sc=False),
    )
    return f(*xi_qs, *xu_qs, gu, gv, ku, kv, ones_i, zf)


def _tc_body(fu_ref, cu_ref, fv_ref, cv_ref, w_ref, hu_ref, hv_ref):
    for f_ref, cnt_ref, out_ref in ((fu_ref, cu_ref, hu_ref),
                                    (fv_ref, cv_ref, hv_ref)):
        acc = jnp.zeros(out_ref.shape, jnp.float32)
        for r in range(R):
            c = cnt_ref[r, :, 0:1]
            rc = jnp.where(c > 0.5, 1.0 / c, 0.0)
            acc = acc + jnp.dot(f_ref[r] * rc, w_ref[r],
                                preferred_element_type=jnp.float32)
        out_ref[...] = acc


def _tc_project(fu, cu, fv, cv, weight):
    blk = 1000
    grid = (NU // blk,)
    fspec = pl.BlockSpec((R, blk, IN_FEAT), lambda g: (0, g, 0))
    cspec = pl.BlockSpec((R, blk, QCOL), lambda g: (0, g, 0))
    ospec = pl.BlockSpec((blk, HID), lambda g: (g, 0))
    return pl.pallas_call(
        _tc_body,
        grid=grid,
        in_specs=[fspec, cspec, fspec, cspec,
                  pl.BlockSpec((R, IN_FEAT, HID), lambda g: (0, 0, 0))],
        out_specs=[ospec, ospec],
        out_shape=[jax.ShapeDtypeStruct((NU, HID), jnp.float32),
                   jax.ShapeDtypeStruct((NI, HID), jnp.float32)],
    )(fu, cu, fv, cv, weight)


def kernel(x_user, x_item, u_s, v_s, rate, weight):
    E = u_s.shape[0]
    nchunk = -(-E // (NS * CHUNK))
    e_pad = NS * nchunk * CHUNK
    pad = e_pad - E

    u_s = u_s.astype(jnp.int32)
    v_s = v_s.astype(jnp.int32)
    rate = rate.astype(jnp.int32)

    key_u = rate * NU + u_s
    key_v = rate * NI + v_s
    trash = jnp.full((pad,), R * NU, jnp.int32)
    zero_idx = jnp.zeros((pad,), jnp.int32)
    ku = jnp.concatenate([key_u, trash]).reshape(NS, nchunk, CHUNK)
    kv = jnp.concatenate([key_v, trash]).reshape(NS, nchunk, CHUNK)
    gu = jnp.concatenate([v_s, zero_idx]).reshape(NS, nchunk, CHUNK)
    gv = jnp.concatenate([u_s, zero_idx]).reshape(NS, nchunk, CHUNK)

    xi_qs = [x_item[:, q * QCOL:(q + 1) * QCOL] for q in range(NQ)]
    xu_qs = [x_user[:, q * QCOL:(q + 1) * QCOL] for q in range(NQ)]
    ones_i = jnp.ones((CHUNK, QCOL), jnp.float32)
    zf = jnp.zeros((ROWS_PT, QCOL), jnp.float32)

    ou, ouc, ov, ovc = _sc_accumulate(
        xi_qs, xu_qs, gu, gv, ku, kv, ones_i, zf)

    h_u, h_v = _tc_project(
        ou.reshape(R, NU, IN_FEAT), ouc.reshape(R, NU, QCOL),
        ov.reshape(R, NI, IN_FEAT), ovc.reshape(R, NI, QCOL), weight)
    return h_u, h_v


# speedup vs baseline: 1.1600x; 1.0000x over previous
import jax
import jax.numpy as jnp
from jax import lax
from jax.experimental import pallas as pl
from jax.experimental.pallas import tpu as pltpu
from jax.experimental.pallas import tpu_sc as plsc

NU = 5000
NI = 5000
R = 5
IN_FEAT = 128
HID = 128
HALF = 64
QCOL = 32
NQ = IN_FEAT // QCOL

NC = 2
NS = 16
CHUNK = 128

ACC_ROWS = R * NU + 88
ROWS_PT = ACC_ROWS // NS


def _sc_mesh():
    return plsc.VectorSubcoreMesh(
        core_axis_name="c", subcore_axis_name="s", num_cores=NC, num_subcores=NS)


NBUF = 6
LAG = 3


def _sc_accumulate(xi_qs, xu_qs, gu, gv, ku, kv, ones_i, zf):
    nchunk = gu.shape[1]
    nq = len(xi_qs)

    def body(*refs):
        xi_t = refs[0:nq]
        xu_t = refs[nq:2 * nq]
        gu_h, gv_h, ku_h, kv_h, ones_h, zf_h = refs[2 * nq:2 * nq + 6]
        ou, ouc, ov, ovc = refs[2 * nq + 6:2 * nq + 10]
        (keys_t, gidx_t, rows_t, ones_t, acc_f, tab_sp,
         sem_g, sem_s) = refs[2 * nq + 10:]
        cid = lax.axis_index("c")
        sid = lax.axis_index("s")
        base = sid * ROWS_PT
        tail_rows = R * NU - (NS - 1) * ROWS_PT

        def dump(src, dst_cols):
            @pl.when(sid < NS - 1)
            def _():
                pltpu.sync_copy(src.at[pl.ds(base, ROWS_PT)],
                                dst_cols(pl.ds(base, ROWS_PT)))

            @pl.when(sid == NS - 1)
            def _():
                pltpu.sync_copy(src.at[pl.ds(base, tail_rows)],
                                dst_cols(pl.ds(base, tail_rows)))

        def run_side(tabs, gidx_hbm, keys_hbm, o, oc):
            pltpu.sync_copy(keys_hbm.at[sid], keys_t)
            pltpu.sync_copy(gidx_hbm.at[sid], gidx_t)
            pltpu.sync_copy(ones_h, ones_t)

            scope_cnt = jax.named_scope("cnt_pass")
            scope_cnt.__enter__()
            pltpu.sync_copy(zf_h, acc_f.at[pl.ds(base, ROWS_PT)])
            plsc.subcore_barrier()

            def cchunk(j, carry):
                pltpu.async_copy(ones_t, acc_f.at[keys_t.at[j]], sem_s, add=True)

                @pl.when(j >= LAG)
                def _():
                    pltpu.make_async_copy(
                        ones_t, acc_f.at[keys_t.at[0]], sem_s).wait()

                return carry

            lax.fori_loop(0, nchunk, cchunk, 0)
            for _ in range(LAG):
                pltpu.make_async_copy(
                    ones_t, acc_f.at[keys_t.at[0]], sem_s).wait()
            plsc.subcore_barrier()
            dump(acc_f, lambda rs: oc.at[rs])
            plsc.subcore_barrier()
            scope_cnt.__exit__(None, None, None)

            for q in range(nq):
                scope_q = jax.named_scope(f"feat_pass{q}")
                scope_q.__enter__()
                tab = tab_sp
                pltpu.sync_copy(zf_h, acc_f.at[pl.ds(base, ROWS_PT)])

                @pl.when(sid == 0)
                def _(q=q):
                    pltpu.sync_copy(tabs[q], tab_sp)

                plsc.subcore_barrier()

                for b in range(LAG):
                    pltpu.async_copy(tab.at[gidx_t.at[b]], rows_t.at[b], sem_g)

                def chunk(j, carry):
                    slot = lax.rem(j, NBUF)
                    pltpu.make_async_copy(
                        tab.at[gidx_t.at[j]], rows_t.at[slot], sem_g).wait()
                    pltpu.async_copy(
                        rows_t.at[slot], acc_f.at[keys_t.at[j]], sem_s, add=True)

                    @pl.when(j + LAG < nchunk)
                    def _():
                        @pl.when(j >= LAG)
                        def _():
                            pltpu.make_async_copy(
                                rows_t.at[0], acc_f.at[keys_t.at[0]],
                                sem_s).wait()

                        pltpu.async_copy(
                            tab.at[gidx_t.at[j + LAG]],
                            rows_t.at[lax.rem(j + LAG, NBUF)], sem_g)

                    return carry

                lax.fori_loop(0, nchunk, chunk, 0)
                for b in range(NBUF):
                    pltpu.make_async_copy(
                        rows_t.at[b], acc_f.at[keys_t.at[0]], sem_s).wait()
                plsc.subcore_barrier()
                dump(acc_f, lambda rs, q=q: o.at[rs, pl.ds(q * QCOL, QCOL)])
                plsc.subcore_barrier()
                scope_q.__exit__(None, None, None)

        @pl.when(cid == 0)
        def _():
            run_side(xi_t, gu_h, ku_h, ou, ouc)

        @pl.when(cid == 1)
        def _():
            run_side(xu_t, gv_h, kv_h, ov, ovc)

    f = pl.kernel(
        body,
        out_type=[
            jax.ShapeDtypeStruct((R * NU, IN_FEAT), jnp.float32),
            jax.ShapeDtypeStruct((R * NU, QCOL), jnp.float32),
            jax.ShapeDtypeStruct((R * NI, IN_FEAT), jnp.float32),
            jax.ShapeDtypeStruct((R * NI, QCOL), jnp.float32),
        ],
        mesh=_sc_mesh(),
        scratch_types=[
            pltpu.VMEM((nchunk, CHUNK), jnp.int32),
            pltpu.VMEM((nchunk, CHUNK), jnp.int32),
            pltpu.VMEM((NBUF, CHUNK, QCOL), jnp.float32),
            pltpu.VMEM((CHUNK, QCOL), jnp.float32),
            pltpu.VMEM_SHARED((ACC_ROWS, QCOL), jnp.float32),
            pltpu.VMEM_SHARED((NU, QCOL), jnp.float32),
            pltpu.SemaphoreType.DMA,
            pltpu.SemaphoreType.DMA,
        ],
        compiler_params=pltpu.CompilerParams(use_tc_tiling_on_sc=False),
    )
    return f(*xi_qs, *xu_qs, gu, gv, ku, kv, ones_i, zf)


def _tc_body(fu_ref, cu_ref, fv_ref, cv_ref, w_ref, hu_ref, hv_ref):
    for f_ref, cnt_ref, out_ref in ((fu_ref, cu_ref, hu_ref),
                                    (fv_ref, cv_ref, hv_ref)):
        acc = jnp.zeros(out_ref.shape, jnp.float32)
        for r in range(R):
            c = cnt_ref[r, :, 0:1]
            rc = jnp.where(c > 0.5, 1.0 / c, 0.0)
            acc = acc + jnp.dot(f_ref[r] * rc, w_ref[r],
                                preferred_element_type=jnp.float32)
        out_ref[...] = acc


def _tc_project(fu, cu, fv, cv, weight):
    blk = 1000
    grid = (NU // blk,)
    fspec = pl.BlockSpec((R, blk, IN_FEAT), lambda g: (0, g, 0))
    cspec = pl.BlockSpec((R, blk, QCOL), lambda g: (0, g, 0))
    ospec = pl.BlockSpec((blk, HID), lambda g: (g, 0))
    return pl.pallas_call(
        _tc_body,
        grid=grid,
        in_specs=[fspec, cspec, fspec, cspec,
                  pl.BlockSpec((R, IN_FEAT, HID), lambda g: (0, 0, 0))],
        out_specs=[ospec, ospec],
        out_shape=[jax.ShapeDtypeStruct((NU, HID), jnp.float32),
                   jax.ShapeDtypeStruct((NI, HID), jnp.float32)],
    )(fu, cu, fv, cv, weight)


def kernel(x_user, x_item, u_s, v_s, rate, weight):
    E = u_s.shape[0]
    nchunk = -(-E // (NS * CHUNK))
    e_pad = NS * nchunk * CHUNK
    pad = e_pad - E

    u_s = u_s.astype(jnp.int32)
    v_s = v_s.astype(jnp.int32)
    rate = rate.astype(jnp.int32)

    key_u = rate * NU + u_s
    key_v = rate * NI + v_s
    trash = jnp.full((pad,), R * NU, jnp.int32)
    zero_idx = jnp.zeros((pad,), jnp.int32)
    ku = jnp.concatenate([key_u, trash]).reshape(NS, nchunk, CHUNK)
    kv = jnp.concatenate([key_v, trash]).reshape(NS, nchunk, CHUNK)
    gu = jnp.concatenate([v_s, zero_idx]).reshape(NS, nchunk, CHUNK)
    gv = jnp.concatenate([u_s, zero_idx]).reshape(NS, nchunk, CHUNK)

    xi_qs = [x_item[:, q * QCOL:(q + 1) * QCOL] for q in range(NQ)]
    xu_qs = [x_user[:, q * QCOL:(q + 1) * QCOL] for q in range(NQ)]
    ones_i = jnp.ones((CHUNK, QCOL), jnp.float32)
    zf = jnp.zeros((ROWS_PT, QCOL), jnp.float32)

    ou, ouc, ov, ovc = _sc_accumulate(
        xi_qs, xu_qs, gu, gv, ku, kv, ones_i, zf)

    h_u, h_v = _tc_project(
        ou.reshape(R, NU, IN_FEAT), ouc.reshape(R, NU, QCOL),
        ov.reshape(R, NI, IN_FEAT), ovc.reshape(R, NI, QCOL), weight)
    return h_u, h_v
